# baseline (speedup 1.0000x reference)
import jax
import jax.numpy as jnp
from jax import lax
from jax.experimental import pallas as pl
from jax.experimental.pallas import tpu as pltpu
from jax.experimental.pallas import tpu_sc as plsc

DIM = 64
NWORKERS = 32
W = 256
ROWS_PER_BLOCK = 8000


def _widen_table(table):
    v = table.shape[0]

    def body(t_ref, o_ref):
        t = t_ref[...]
        o_ref[:, :DIM] = t
        o_ref[:, DIM:] = t

    return pl.pallas_call(
        body,
        grid=(v // ROWS_PER_BLOCK,),
        in_specs=[pl.BlockSpec((ROWS_PER_BLOCK, DIM), lambda i: (i, 0))],
        out_specs=pl.BlockSpec((ROWS_PER_BLOCK, 2 * DIM), lambda i: (i, 0)),
        out_shape=jax.ShapeDtypeStruct((v, 2 * DIM), table.dtype),
        compiler_params=pltpu.CompilerParams(
            dimension_semantics=("parallel",),
        ),
    )(table)


def kernel(x, table):
    B, S = x.shape
    n = B * S
    idx = x.reshape(n)
    b_per_w = n // NWORKERS
    steps = b_per_w // W

    wide = _widen_table(table)
    mesh = plsc.VectorSubcoreMesh(core_axis_name="c", subcore_axis_name="s")

    @pl.kernel(
        out_type=jax.ShapeDtypeStruct((n, 2 * DIM), table.dtype),
        mesh=mesh,
        scratch_types=[
            pltpu.VMEM((W,), jnp.int32),
            pltpu.VMEM((W, 2 * DIM), jnp.float32),
            pltpu.SemaphoreType.DMA,
        ],
    )
    def gather_kernel(wide_hbm, i_hbm, o_hbm, idx_v, gbuf, sem):
        wid = lax.axis_index("s") * 2 + lax.axis_index("c")

        @pl.loop(0, steps)
        def _(c):
            base = wid * b_per_w + c * W
            pltpu.sync_copy(i_hbm.at[pl.ds(base, W)], idx_v)
            pltpu.async_copy(wide_hbm.at[idx_v], gbuf, sem).wait()
            pltpu.sync_copy(gbuf, o_hbm.at[pl.ds(base, W)])

    out = gather_kernel(wide, idx)
    return out[:, :DIM].reshape(B, S, DIM)
